# baseline (speedup 1.0000x reference)
import functools

import jax
import jax.numpy as jnp
import numpy as np
from jax import lax
from jax.experimental import pallas as pl
from jax.experimental.pallas import tpu as pltpu
from jax.experimental.shard_map import shard_map
from jax.sharding import Mesh, NamedSharding, PartitionSpec as P

_DOT_LAST_LAST = (((1,), (1,)), ((), ()))

_VMEM_LIMIT = 60 * 1024 * 1024


def _round_up(v, m):
    return (v + m - 1) // m * m


def _body(x_ref, w_ref, m_ref, o_ref, wg_ref):
    @pl.when(pl.program_id(0) == 0)
    def _():
        wg_ref[...] = jnp.tanh(w_ref[...]) * jax.nn.sigmoid(m_ref[...])

    o_ref[...] = lax.dot_general(
        x_ref[...], wg_ref[...],
        dimension_numbers=_DOT_LAST_LAST,
        preferred_element_type=jnp.float32,
        precision=lax.Precision.DEFAULT,
    )


def _nac_fused(x, w_, m_, tm):
    B, K = x.shape
    N = w_.shape[0]
    tm = min(tm, _round_up(B, 8))
    Bp = _round_up(B, tm)
    if Bp != B:
        x = jnp.pad(x, ((0, Bp - B), (0, 0)))

    wfull = pl.BlockSpec((N, K), lambda i: (0, 0))
    yp = pl.pallas_call(
        _body,
        out_shape=jax.ShapeDtypeStruct((Bp, N), jnp.float32),
        grid=(Bp // tm,),
        in_specs=[
            pl.BlockSpec((tm, K), lambda i: (i, 0)),
            wfull,
            wfull,
        ],
        out_specs=pl.BlockSpec((tm, N), lambda i: (i, 0)),
        scratch_shapes=[pltpu.VMEM((N, K), jnp.float32)],
        compiler_params=pltpu.CompilerParams(
            dimension_semantics=("arbitrary",),
            vmem_limit_bytes=_VMEM_LIMIT,
        ),
    )(x, w_, m_)
    return yp[:B] if Bp != B else yp


def kernel(x, w_, m_):
    assert x.ndim == 2 and w_.shape == m_.shape and x.shape[1] == w_.shape[1]
    B = x.shape[0]
    devs = jax.devices()
    if len(devs) >= 2 and B % 2 == 0 and B >= 16:
        mesh = Mesh(np.array(devs[:2]), ("d",))
        xs = lax.with_sharding_constraint(
            x, NamedSharding(mesh, P("d", None)))
        ws = lax.with_sharding_constraint(w_, NamedSharding(mesh, P()))
        ms = lax.with_sharding_constraint(m_, NamedSharding(mesh, P()))
        fn = shard_map(
            functools.partial(_nac_fused, tm=2048),
            mesh=mesh,
            in_specs=(P("d", None), P(), P()),
            out_specs=P("d", None),
            check_rep=False,
        )
        return fn(xs, ws, ms)
    return _nac_fused(x, w_, m_, tm=2048)


# speedup vs baseline: 12.6744x; 12.6744x over previous
import functools

import jax
import jax.numpy as jnp
from jax import lax
from jax.experimental import pallas as pl
from jax.experimental.pallas import tpu as pltpu

_DOT_LAST_LAST = (((1,), (1,)), ((), ()))

_VMEM_LIMIT = 60 * 1024 * 1024


def _round_up(v, m):
    return (v + m - 1) // m * m


def _body(x_ref, w_ref, m_ref, o_ref, wg_ref):
    @pl.when(pl.program_id(0) == 0)
    def _():
        wg_ref[...] = jnp.tanh(w_ref[...]) * jax.nn.sigmoid(m_ref[...])

    o_ref[...] = lax.dot_general(
        x_ref[...], wg_ref[...],
        dimension_numbers=_DOT_LAST_LAST,
        preferred_element_type=jnp.float32,
        precision=lax.Precision.DEFAULT,
    )


def _nac_fused(x, w_, m_, tm):
    B, K = x.shape
    N = w_.shape[0]
    tm = min(tm, _round_up(B, 8))
    Bp = _round_up(B, tm)
    if Bp != B:
        x = jnp.pad(x, ((0, Bp - B), (0, 0)))

    wfull = pl.BlockSpec((N, K), lambda i: (0, 0))
    yp = pl.pallas_call(
        _body,
        out_shape=jax.ShapeDtypeStruct((Bp, N), jnp.float32),
        grid=(Bp // tm,),
        in_specs=[
            pl.BlockSpec((tm, K), lambda i: (i, 0)),
            wfull,
            wfull,
        ],
        out_specs=pl.BlockSpec((tm, N), lambda i: (i, 0)),
        scratch_shapes=[pltpu.VMEM((N, K), jnp.float32)],
        compiler_params=pltpu.CompilerParams(
            dimension_semantics=("arbitrary",),
            vmem_limit_bytes=_VMEM_LIMIT,
        ),
    )(x, w_, m_)
    return yp[:B] if Bp != B else yp


def kernel(x, w_, m_):
    assert x.ndim == 2 and w_.shape == m_.shape and x.shape[1] == w_.shape[1]
    return _nac_fused(x, w_, m_, tm=2048)
